# baseline (speedup 1.0000x reference)
import jax
import jax.numpy as jnp
from jax.experimental import pallas as pl
from jax.experimental.pallas import tpu as pltpu

P = 16
S = 2
C = 3
H = W = 512
OH = OW = (H - P) // S + 1
L = OH * OW
F = C * P * P


def _group_block(rows8):
    T = jnp.swapaxes(rows8, 0, 1)
    Q = T.reshape(W // 2, 2, 8)
    Q = jnp.swapaxes(Q, 1, 2).reshape(W // 2, 16)
    rep = jnp.broadcast_to(
        Q.reshape(W // 2, 8, 1, 2), (W // 2, 8, 8, 2)
    ).reshape(W // 2, 128)
    lane_m = (jax.lax.broadcasted_iota(jnp.int32, (OW, 128), 1) // 2) % 8
    acc = jnp.zeros((OW, 128), jnp.float32)
    for m in range(8):
        acc = acc + jnp.where(lane_m == m, rep[m:m + OW, :], 0.0)
    return acc


def _window16(x_ref, i, c):
    q = jnp.minimum(i // 4, (H - 24) // 8)
    base = pl.multiple_of(8 * q, 8)
    rows24 = x_ref[0, c, pl.ds(base, 24), :]
    r = S * i - 8 * q
    return jax.lax.switch(
        r // 2, [lambda k=k: rows24[2 * k:2 * k + P, :] for k in range(5)]
    )


def _chan_block(x_ref, i, c):
    rows16 = _window16(x_ref, i, c)
    return jnp.concatenate(
        [_group_block(rows16[0:8, :]), _group_block(rows16[8:16, :])], axis=1)


def _body(x1_ref, x2_ref, o1_ref, o2_ref):
    i = pl.program_id(0)
    c = pl.program_id(1)
    o1_ref[0] = _chan_block(x1_ref, i, c)
    o2_ref[0] = _chan_block(x2_ref, i, c)


def kernel(input_1, input_2):
    full = pl.BlockSpec((1, C, H, W), lambda i, c: (0, 0, 0, 0))
    outb = pl.BlockSpec((1, OW, 256), lambda i, c: (i, 0, c))
    o1, o2 = pl.pallas_call(
        _body,
        grid=(OH, C),
        in_specs=[full, full],
        out_specs=[outb, outb],
        out_shape=[
            jax.ShapeDtypeStruct((OH, OW, F), jnp.float32),
            jax.ShapeDtypeStruct((OH, OW, F), jnp.float32),
        ],
    )(input_1, input_2)
    return o1.reshape(L, F), o2.reshape(L, F)


# speedup vs baseline: 5.5354x; 5.5354x over previous
import jax
import jax.numpy as jnp
from jax.experimental import pallas as pl
from jax.experimental.pallas import tpu as pltpu

P = 16
S = 2
C = 3
H = W = 512
OH = OW = (H - P) // S + 1
L = OH * OW
F = C * P * P
D = W // 2


def _window16(ref, i, img_c):
    q = jnp.minimum(i // 4, (H - 24) // 8)
    base = pl.multiple_of(8 * q, 8)
    rows24 = ref[img_c, pl.ds(base, 24), :]
    r = S * i - 8 * q
    return jax.lax.switch(
        r // 2, [lambda k=k: rows24[2 * k:2 * k + P, :] for k in range(5)]
    )


def _rows48(ref, i):
    return jnp.concatenate([_window16(ref, i, c) for c in range(C)], axis=0)


def _emit_block(ed, od, o_ref):
    par = jax.lax.broadcasted_iota(jnp.int32, (8, D), 0) % 2
    pp = jax.lax.broadcasted_iota(jnp.int32, (8, OW), 0) // 2
    ppm = pp % 2
    for g in range(F // 128):
        tiles = []
        for t8 in range(16 * g, 16 * (g + 1)):
            t, half = divmod(t8, 2)
            if half == 0:
                v0 = jnp.where(par == 1,
                               jnp.broadcast_to(od[t:t + 1, :], (8, D)),
                               jnp.broadcast_to(ed[t:t + 1, :], (8, D)))
            mb = 4 * half
            s = [v0[:, mb + k: mb + k + OW] for k in range(4)]
            lo = jnp.where(ppm == 1, s[1], s[0])
            hi = jnp.where(ppm == 1, s[3], s[2])
            tiles.append(jnp.where(pp >= 2, hi, lo))
        grp = jnp.concatenate(tiles, axis=0)
        o_ref[0, :, 128 * g:128 * (g + 1)] = jnp.swapaxes(grp, 0, 1)


def _body(xe1, xo1, xe2, xo2, o1_ref, o2_ref):
    i = pl.program_id(0)
    _emit_block(_rows48(xe1, i), _rows48(xo1, i), o1_ref)
    _emit_block(_rows48(xe2, i), _rows48(xo2, i), o2_ref)


def kernel(input_1, input_2):
    xe1, xo1 = input_1[0, :, :, 0::2], input_1[0, :, :, 1::2]
    xe2, xo2 = input_2[0, :, :, 0::2], input_2[0, :, :, 1::2]
    full = pl.BlockSpec((C, H, D), lambda i: (0, 0, 0))
    outb = pl.BlockSpec((1, OW, F), lambda i: (i, 0, 0))
    o1, o2 = pl.pallas_call(
        _body,
        grid=(OH,),
        in_specs=[full, full, full, full],
        out_specs=[outb, outb],
        out_shape=[
            jax.ShapeDtypeStruct((OH, OW, F), jnp.float32),
            jax.ShapeDtypeStruct((OH, OW, F), jnp.float32),
        ],
    )(xe1, xo1, xe2, xo2)
    return o1.reshape(L, F), o2.reshape(L, F)


# speedup vs baseline: 10.2499x; 1.8517x over previous
import numpy as np
import jax
import jax.numpy as jnp
from jax.experimental import pallas as pl
from jax.experimental.pallas import tpu as pltpu

P = 16
S = 2
C = 3
H = W = 512
OH = OW = (H - P) // S + 1
L = OH * OW
F = C * P * P
D = W // 2
NT = C * P


def _proj_matrix():
    p = np.zeros((2 * NT, F), np.float32)
    for t in range(NT):
        for v in range(2):
            for m in range(8):
                p[48 * v + t, 16 * t + 2 * m + v] = 1.0
    return jnp.asarray(p, jnp.bfloat16)


def _window16(ref, i, img_c):
    q = jnp.minimum(i // 4, (H - 24) // 8)
    base = pl.multiple_of(8 * q, 8)
    rows24 = ref[img_c, pl.ds(base, 24), :]
    r = S * i - 8 * q
    return jax.lax.switch(
        r // 2, [lambda k=k: rows24[2 * k:2 * k + P, :] for k in range(5)]
    )


def _rows48(ref, i):
    return jnp.concatenate([_window16(ref, i, c) for c in range(C)], axis=0)


def _emit_block(xe, xo, p_ref, o_ref, i):
    eo96 = jnp.concatenate([_rows48(xe, i), _rows48(xo, i)], axis=0)
    tr = jnp.swapaxes(eo96, 0, 1)
    hi = tr.astype(jnp.bfloat16)
    lo = (tr - hi.astype(jnp.float32)).astype(jnp.bfloat16)
    pm = p_ref[...]
    dn = (((1,), (0,)), ((), ()))
    rep = (jax.lax.dot_general(hi, pm, dn, preferred_element_type=jnp.float32)
           + jax.lax.dot_general(lo, pm, dn, preferred_element_type=jnp.float32))
    lm = (jax.lax.broadcasted_iota(jnp.int32, (1, F), 1) // 2) % 8
    acc = jnp.where(lm == 0, rep[0:OW, :], 0.0)
    for m in range(1, 8):
        acc = jnp.where(lm == m, rep[m:m + OW, :], acc)
    o_ref[0] = acc


def _body(xe1, xo1, xe2, xo2, p_ref, o1_ref, o2_ref):
    i = pl.program_id(0)
    _emit_block(xe1, xo1, p_ref, o1_ref, i)
    _emit_block(xe2, xo2, p_ref, o2_ref, i)


def kernel(input_1, input_2):
    xe1, xo1 = input_1[0, :, :, 0::2], input_1[0, :, :, 1::2]
    xe2, xo2 = input_2[0, :, :, 0::2], input_2[0, :, :, 1::2]
    proj = _proj_matrix()
    full = pl.BlockSpec((C, H, D), lambda i: (0, 0, 0))
    pspec = pl.BlockSpec((2 * NT, F), lambda i: (0, 0))
    outb = pl.BlockSpec((1, OW, F), lambda i: (i, 0, 0))
    o1, o2 = pl.pallas_call(
        _body,
        grid=(OH,),
        in_specs=[full, full, full, full, pspec],
        out_specs=[outb, outb],
        out_shape=[
            jax.ShapeDtypeStruct((OH, OW, F), jnp.float32),
            jax.ShapeDtypeStruct((OH, OW, F), jnp.float32),
        ],
    )(xe1, xo1, xe2, xo2, proj)
    return o1.reshape(L, F), o2.reshape(L, F)


# speedup vs baseline: 15.0186x; 1.4652x over previous
import numpy as np
import jax
import jax.numpy as jnp
from jax.experimental import pallas as pl
from jax.experimental.pallas import tpu as pltpu

P = 16
S = 2
C = 3
H = W = 512
OH = OW = (H - P) // S + 1
L = OH * OW
F = C * P * P
D = W // 2
NT = C * P


def _proj_matrix():
    p = np.zeros((2 * NT, F), np.float32)
    for t in range(NT):
        for v in range(2):
            for m in range(8):
                p[48 * v + t, 16 * t + 2 * m + v] = 1.0
    return jnp.asarray(p, jnp.bfloat16)


def _window16(ref, i, img_c):
    q = jnp.minimum(i // 4, (H - 24) // 8)
    base = pl.multiple_of(8 * q, 8)
    rows24 = ref[img_c, pl.ds(base, 24), :]
    r = S * i - 8 * q
    return jax.lax.switch(
        r // 2, [lambda k=k: rows24[2 * k:2 * k + P, :] for k in range(5)]
    )


def _rows48(ref, i):
    return jnp.concatenate([_window16(ref, i, c) for c in range(C)], axis=0)


def _make_block(xe, xo, pm, i):
    eo96 = jnp.concatenate([_rows48(xe, i), _rows48(xo, i)], axis=0)
    tr = jnp.swapaxes(eo96, 0, 1)
    hi = tr.astype(jnp.bfloat16)
    lo = (tr - hi.astype(jnp.float32)).astype(jnp.bfloat16)
    dn = (((1,), (0,)), ((), ()))
    rep = (jax.lax.dot_general(hi, pm, dn, preferred_element_type=jnp.float32)
           + jax.lax.dot_general(lo, pm, dn, preferred_element_type=jnp.float32))
    lm = (jax.lax.broadcasted_iota(jnp.int32, (1, F), 1) // 2) % 8
    acc = jnp.where(lm == 0, rep[0:OW, :], 0.0)
    for m in range(1, 8):
        acc = jnp.where(lm == m, rep[m:m + OW, :], acc)
    return acc


OHB = 8
NB = (OH + OHB - 1) // OHB


def _body(xe1, xo1, xe2, xo2, p_ref, o1_ref, o2_ref):
    b = pl.program_id(0)
    pm = p_ref[...]
    for k in range(OHB):
        i = jnp.minimum(OHB * b + k, OH - 1)
        o1_ref[pl.ds(OW * k, OW), :] = _make_block(xe1, xo1, pm, i)
        o2_ref[pl.ds(OW * k, OW), :] = _make_block(xe2, xo2, pm, i)


def kernel(input_1, input_2):
    xe1, xo1 = input_1[0, :, :, 0::2], input_1[0, :, :, 1::2]
    xe2, xo2 = input_2[0, :, :, 0::2], input_2[0, :, :, 1::2]
    proj = _proj_matrix()
    full = pl.BlockSpec((C, H, D), lambda b: (0, 0, 0))
    pspec = pl.BlockSpec((2 * NT, F), lambda b: (0, 0))
    outb = pl.BlockSpec((OHB * OW, F), lambda b: (b, 0))
    o1, o2 = pl.pallas_call(
        _body,
        grid=(NB,),
        in_specs=[full, full, full, full, pspec],
        out_specs=[outb, outb],
        out_shape=[
            jax.ShapeDtypeStruct((L, F), jnp.float32),
            jax.ShapeDtypeStruct((L, F), jnp.float32),
        ],
    )(xe1, xo1, xe2, xo2, proj)
    return o1, o2


# speedup vs baseline: 19.0937x; 1.2713x over previous
import numpy as np
import jax
import jax.numpy as jnp
from jax import lax
from jax.experimental import pallas as pl
from jax.experimental.pallas import tpu as pltpu
from jax.experimental.pallas import tpu_sc as plsc

P = 16
S = 2
C = 3
H = W = 512
OH = OW = (H - P) // S + 1
L = OH * OW
F = C * P * P
D = W // 2
NT = C * P


def _proj_matrix():
    p = np.zeros((2 * NT, F), np.float32)
    for t in range(NT):
        for v in range(2):
            for m in range(8):
                p[48 * v + t, 16 * t + 2 * m + v] = 1.0
    return jnp.asarray(p, jnp.bfloat16)


def _window16(ref, i, img_c):
    q = jnp.minimum(i // 4, (H - 24) // 8)
    base = pl.multiple_of(8 * q, 8)
    rows24 = ref[img_c, pl.ds(base, 24), :]
    r = S * i - 8 * q
    return jax.lax.switch(
        r // 2, [lambda k=k: rows24[2 * k:2 * k + P, :] for k in range(5)]
    )


def _rows48(ref, i):
    return jnp.concatenate([_window16(ref, i, c) for c in range(C)], axis=0)


def _make_block(xe, xo, pm, i):
    eo96 = jnp.concatenate([_rows48(xe, i), _rows48(xo, i)], axis=0)
    tr = jnp.swapaxes(eo96, 0, 1)
    hi = tr.astype(jnp.bfloat16)
    lo = (tr - hi.astype(jnp.float32)).astype(jnp.bfloat16)
    dn = (((1,), (0,)), ((), ()))
    rep = (jax.lax.dot_general(hi, pm, dn, preferred_element_type=jnp.float32)
           + jax.lax.dot_general(lo, pm, dn, preferred_element_type=jnp.float32))
    lm = (jax.lax.broadcasted_iota(jnp.int32, (1, F), 1) // 2) % 8
    acc = jnp.where(lm == 0, rep[0:OW, :], 0.0)
    for m in range(1, 8):
        acc = jnp.where(lm == m, rep[m:m + OW, :], acc)
    return acc


OHB = 8
NB = (OH + OHB - 1) // OHB


def _tc_body(xe, xo, p_ref, o_ref):
    b = pl.program_id(0)
    pm = p_ref[...]
    for k in range(OHB):
        i = jnp.minimum(OHB * b + k, OH - 1)
        o_ref[pl.ds(OW * k, OW), :] = _make_block(xe, xo, pm, i)


def _tc_unfold(image):
    xe, xo = image[0, :, :, 0::2], image[0, :, :, 1::2]
    proj = _proj_matrix()
    full = pl.BlockSpec((C, H, D), lambda b: (0, 0, 0))
    pspec = pl.BlockSpec((2 * NT, F), lambda b: (0, 0))
    outb = pl.BlockSpec((OHB * OW, F), lambda b: (b, 0))
    return pl.pallas_call(
        _tc_body,
        grid=(NB,),
        in_specs=[full, full, pspec],
        out_specs=outb,
        out_shape=jax.ShapeDtypeStruct((L, F), jnp.float32),
    )(xe, xo, proj)


NWORK = 32
OH_PER_W = (OH + NWORK - 1) // NWORK
CHUNK = 83


def _sc_unfold(image):
    mesh = plsc.VectorSubcoreMesh(core_axis_name="c", subcore_axis_name="s")

    @pl.kernel(
        mesh=mesh,
        out_type=jax.ShapeDtypeStruct((L, F), jnp.float32),
        compiler_params=pltpu.CompilerParams(use_tc_tiling_on_sc=False),
        scratch_types=[
            pltpu.VMEM((C, P, W), jnp.float32),
            pltpu.VMEM((CHUNK, F), jnp.float32),
            pltpu.SemaphoreType.DMA,
        ],
    )
    def sc_kernel(x_hbm, o_hbm, w_ref, buf_ref, sem):
        wid = lax.axis_index("s") * 2 + lax.axis_index("c")
        for j in range(OH_PER_W):
            oh = NWORK * j + wid

            @pl.when(oh < OH)
            def _():
                pltpu.async_copy(
                    x_hbm.at[:, pl.ds(S * oh, P), :], w_ref, sem).wait()
                for s in range(OH // CHUNK):
                    @pl.loop(0, CHUNK)
                    def _(owl):
                        ow = CHUNK * s + owl
                        for t in range(NT):
                            c, kh = divmod(t, P)
                            buf_ref[owl, pl.ds(P * t, P)] = (
                                w_ref[c, kh, pl.ds(S * ow, P)])
                    pltpu.async_copy(
                        buf_ref,
                        o_hbm.at[pl.ds(OW * oh + CHUNK * s, CHUNK), :],
                        sem).wait()

    return sc_kernel(image[0])


def kernel(input_1, input_2):
    return _tc_unfold(input_1), _sc_unfold(input_2)


# speedup vs baseline: 19.5221x; 1.0224x over previous
import numpy as np
import jax
import jax.numpy as jnp
from jax import lax
from jax.experimental import pallas as pl
from jax.experimental.pallas import tpu as pltpu
from jax.experimental.pallas import tpu_sc as plsc

P = 16
S = 2
C = 3
H = W = 512
OH = OW = (H - P) // S + 1
L = OH * OW
F = C * P * P
D = W // 2
NT = C * P


def _proj_matrix():
    p = np.zeros((2 * NT, F), np.float32)
    for t in range(NT):
        for v in range(2):
            for m in range(8):
                p[48 * v + t, 16 * t + 2 * m + v] = 1.0
    return jnp.asarray(p, jnp.bfloat16)


def _window16(ref, i, img_c):
    q = jnp.minimum(i // 4, (H - 24) // 8)
    base = pl.multiple_of(8 * q, 8)
    rows24 = ref[img_c, pl.ds(base, 24), :]
    r = S * i - 8 * q
    return jax.lax.switch(
        r // 2, [lambda k=k: rows24[2 * k:2 * k + P, :] for k in range(5)]
    )


def _rows48(ref, i):
    return jnp.concatenate([_window16(ref, i, c) for c in range(C)], axis=0)


def _make_block(xe, xo, pm, i):
    eo96 = jnp.concatenate([_rows48(xe, i), _rows48(xo, i)], axis=0)
    tr = jnp.swapaxes(eo96, 0, 1)
    hi = tr.astype(jnp.bfloat16)
    lo = (tr - hi.astype(jnp.float32)).astype(jnp.bfloat16)
    dn = (((1,), (0,)), ((), ()))
    rep = (jax.lax.dot_general(hi, pm, dn, preferred_element_type=jnp.float32)
           + jax.lax.dot_general(lo, pm, dn, preferred_element_type=jnp.float32))
    lm = (jax.lax.broadcasted_iota(jnp.int32, (1, F), 1) // 2) % 8
    acc = jnp.where(lm == 0, rep[0:OW, :], 0.0)
    for m in range(1, 8):
        acc = jnp.where(lm == m, rep[m:m + OW, :], acc)
    return acc


OHB = 8
NB = (OH + OHB - 1) // OHB


def _tc_body(xe, xo, p_ref, o_ref):
    b = pl.program_id(0)
    pm = p_ref[...]
    for k in range(OHB):
        i = jnp.minimum(OHB * b + k, OH - 1)
        o_ref[pl.ds(OW * k, OW), :] = _make_block(xe, xo, pm, i)


def _tc_unfold(image):
    xe, xo = image[0, :, :, 0::2], image[0, :, :, 1::2]
    proj = _proj_matrix()
    full = pl.BlockSpec((C, H, D), lambda b: (0, 0, 0))
    pspec = pl.BlockSpec((2 * NT, F), lambda b: (0, 0))
    outb = pl.BlockSpec((OHB * OW, F), lambda b: (b, 0))
    return pl.pallas_call(
        _tc_body,
        grid=(NB,),
        in_specs=[full, full, pspec],
        out_specs=outb,
        out_shape=jax.ShapeDtypeStruct((L, F), jnp.float32),
    )(xe, xo, proj)


NWORK = 32
OH_PER_W = (OH + NWORK - 1) // NWORK
CHUNK = 48
SLABS = [(s * CHUNK, min(CHUNK, OW - s * CHUNK))
         for s in range((OW + CHUNK - 1) // CHUNK)]


def _sc_unfold(image):
    mesh = plsc.VectorSubcoreMesh(core_axis_name="c", subcore_axis_name="s")

    @pl.kernel(
        mesh=mesh,
        out_type=jax.ShapeDtypeStruct((L, F), jnp.float32),
        compiler_params=pltpu.CompilerParams(use_tc_tiling_on_sc=False),
        scratch_types=[
            pltpu.VMEM((C, P, W), jnp.float32),
            pltpu.VMEM((CHUNK, F), jnp.float32),
            pltpu.VMEM((CHUNK, F), jnp.float32),
            pltpu.SemaphoreType.DMA,
            pltpu.SemaphoreType.DMA,
            pltpu.SemaphoreType.DMA,
        ],
    )
    def sc_kernel(x_hbm, o_hbm, w_ref, buf0, buf1, sem0, sem1, wsem):
        wid = lax.axis_index("s") * 2 + lax.axis_index("c")
        bufs, sems = [buf0, buf1], [sem0, sem1]

        @pl.loop(0, OH_PER_W)
        def _(j):
            oh = jnp.minimum(NWORK * j + wid, OH - 1)
            pltpu.async_copy(
                x_hbm.at[:, pl.ds(S * oh, P), :], w_ref, wsem).wait()
            pending = [None, None]
            for k, (ow0, n) in enumerate(SLABS):
                b = k % 2
                if pending[b] is not None:
                    pending[b].wait()

                @pl.loop(0, n)
                def _(owl, ow0=ow0, b=b):
                    ow = ow0 + owl
                    for t in range(NT):
                        c, kh = divmod(t, P)
                        bufs[b][owl, pl.ds(P * t, P)] = (
                            w_ref[c, kh, pl.ds(S * ow, P)])
                cp = pltpu.make_async_copy(
                    bufs[b].at[pl.ds(0, n), :],
                    o_hbm.at[pl.ds(OW * oh + ow0, n), :],
                    sems[b])
                cp.start()
                pending[b] = cp
            for b in range(2):
                if pending[b] is not None:
                    pending[b].wait()

    return sc_kernel(image[0])


def kernel(input_1, input_2):
    return _tc_unfold(input_1), _sc_unfold(input_2)
